# baseline (speedup 1.0000x reference)
import functools

import jax
import jax.numpy as jnp
from jax import lax
from jax.experimental import pallas as pl
from jax.experimental.pallas import tpu as pltpu
from jax.experimental.pallas import tpu_sc as plsc

_NC = 2
_NS = 16
_NW = _NC * _NS
_LANES = 16


def _bag_mean_sc(gidx, goff, emb2):
    B, L = gidx.shape
    D2 = emb2.shape[1]
    D = D2 // 2
    bags_w = B // _NW
    mesh = plsc.VectorSubcoreMesh(core_axis_name="c", subcore_axis_name="s")

    @functools.partial(
        pl.kernel,
        out_type=jax.ShapeDtypeStruct((B, D), jnp.float32),
        mesh=mesh,
        compiler_params=pltpu.CompilerParams(needs_layout_passes=False),
        scratch_types=[
            pltpu.VMEM((bags_w, L), jnp.int32),
            pltpu.VMEM((bags_w, L), jnp.int32),
            pltpu.VMEM((L, D2), jnp.float32),
            pltpu.VMEM((L, D2), jnp.float32),
            pltpu.VMEM((bags_w, D), jnp.float32),
            pltpu.SemaphoreType.DMA,
            pltpu.SemaphoreType.DMA,
        ],
    )
    def k(emb_hbm, gidx_hbm, goff_hbm, out_hbm,
          idx_v, off_v, rows0, rows1, out_v, sem0, sem1):
        wid = lax.axis_index("s") * _NC + lax.axis_index("c")
        row0 = wid * bags_w
        pltpu.sync_copy(gidx_hbm.at[pl.ds(row0, bags_w)], idx_v)
        pltpu.sync_copy(goff_hbm.at[pl.ds(row0, bags_w)], off_v)

        scale = 1.0 / float(L)
        lane = jnp.arange(_LANES, dtype=jnp.int32)
        col_base = [lane + d * _LANES for d in range(D // _LANES)]

        def accum(rows_ref, b):
            bvec = jnp.full((_LANES,), b, dtype=jnp.int32)
            accs = [jnp.zeros((_LANES,), jnp.float32) for _ in range(D // _LANES)]
            for j in range(L):
                off = plsc.load_gather(
                    off_v, [bvec, jnp.full((_LANES,), j, dtype=jnp.int32)])
                jvec = jnp.full((_LANES,), j, dtype=jnp.int32)
                for d in range(D // _LANES):
                    val = plsc.load_gather(rows_ref, [jvec, off + col_base[d]])
                    accs[d] = accs[d] + val
            for d in range(D // _LANES):
                out_v[b, pl.ds(d * _LANES, _LANES)] = accs[d] * scale

        def body(g, carry):
            b0 = 2 * g
            b1 = b0 + 1
            c0 = pltpu.async_copy(emb_hbm.at[idx_v.at[b0]], rows0, sem0)
            c1 = pltpu.async_copy(emb_hbm.at[idx_v.at[b1]], rows1, sem1)
            c0.wait()
            accum(rows0, b0)
            c1.wait()
            accum(rows1, b1)
            return carry

        lax.fori_loop(0, bags_w // 2, body, 0)
        pltpu.sync_copy(out_v, out_hbm.at[pl.ds(row0, bags_w)])

    return k(emb2, gidx, goff)


def _classifier_tc(bag, fc_weight, fc_bias2d):
    B, D = bag.shape
    C = fc_weight.shape[0]
    blk = 512

    def body(x_ref, w_ref, b_ref, o_ref):
        x = x_ref[...]
        w = w_ref[...]
        logits = lax.dot_general(
            x, w, (((1,), (1,)), ((), ())), preferred_element_type=jnp.float32
        )
        logits = logits + b_ref[...]
        m = jnp.max(logits, axis=1, keepdims=True)
        e = jnp.exp(logits - m)
        lse = jnp.log(jnp.sum(e, axis=1, keepdims=True)) + m
        o_ref[...] = logits - lse

    return pl.pallas_call(
        body,
        grid=(B // blk,),
        in_specs=[
            pl.BlockSpec((blk, D), lambda i: (i, 0)),
            pl.BlockSpec((C, D), lambda i: (0, 0)),
            pl.BlockSpec((1, C), lambda i: (0, 0)),
        ],
        out_specs=pl.BlockSpec((blk, C), lambda i: (i, 0)),
        out_shape=jax.ShapeDtypeStruct((B, C), jnp.float32),
    )(bag, fc_weight, fc_bias2d)


def kernel(text, offsets, emb_weight, fc_weight, fc_bias):
    B = offsets.shape[0]
    T = text.shape[0]
    L = T // B
    V, D = emb_weight.shape
    C = fc_weight.shape[0]
    text2d = text.reshape(B, L)
    emb2 = emb_weight.reshape(V // 2, 2 * D)
    gidx = lax.shift_right_logical(text2d, 1)
    goff = (text2d & 1) * D
    bag = _bag_mean_sc(gidx, goff, emb2)
    return _classifier_tc(bag, fc_weight, fc_bias.reshape(1, C))


# speedup vs baseline: 1.9682x; 1.9682x over previous
import functools

import jax
import jax.numpy as jnp
from jax import lax
from jax.experimental import pallas as pl
from jax.experimental.pallas import tpu as pltpu
from jax.experimental.pallas import tpu_sc as plsc

_NC = 2
_NS = 16
_NW = _NC * _NS
_LANES = 16

_TB = 2048
_NSEG = 4


def _project_tc(embT, fc_weight, V):
    D = embT.shape[0]
    C = fc_weight.shape[0]
    nblk = -(-V // (_NSEG * _TB))
    R = nblk * _TB

    def body(x0_ref, x1_ref, x2_ref, x3_ref, w_ref, o_ref):
        w = w_ref[...]
        z = jnp.zeros((_TB, 32 - C), jnp.float32)
        outs = []
        for x_ref in (x0_ref, x1_ref, x2_ref, x3_ref):
            x = x_ref[...]
            o = lax.dot_general(
                x, w, (((0,), (1,)), ((), ())),
                preferred_element_type=jnp.float32)
            outs.extend([o, z])
        o_ref[...] = jnp.concatenate(outs, axis=1)

    last_blk = -(-V // _TB) - 1
    specs = [
        pl.BlockSpec((D, _TB), functools.partial(
            lambda s, i: (0, jnp.minimum(s * nblk + i, last_blk)), s))
        for s in range(_NSEG)
    ]
    specs.append(pl.BlockSpec((C, D), lambda i: (0, 0)))
    return pl.pallas_call(
        body,
        grid=(nblk,),
        in_specs=specs,
        out_specs=pl.BlockSpec((_TB, 128), lambda i: (i, 0)),
        out_shape=jax.ShapeDtypeStruct((R, 128), jnp.float32),
    )(embT, embT, embT, embT, fc_weight), R


def _bag_sum_sc(gidx, goff, P):
    B, L = gidx.shape
    bags_w = B // _NW
    mesh = plsc.VectorSubcoreMesh(core_axis_name="c", subcore_axis_name="s")

    @functools.partial(
        pl.kernel,
        out_type=jax.ShapeDtypeStruct((B, 2 * _LANES), jnp.float32),
        mesh=mesh,
        compiler_params=pltpu.CompilerParams(needs_layout_passes=False),
        scratch_types=[
            pltpu.VMEM((bags_w, L), jnp.int32),
            pltpu.VMEM((bags_w, L), jnp.int32),
            pltpu.VMEM((L, 128), jnp.float32),
            pltpu.VMEM((L, 128), jnp.float32),
            pltpu.VMEM((bags_w, 2 * _LANES), jnp.float32),
            pltpu.SemaphoreType.DMA,
            pltpu.SemaphoreType.DMA,
        ],
    )
    def k(p_hbm, gidx_hbm, goff_hbm, out_hbm,
          idx_v, off_v, rows0, rows1, out_v, sem0, sem1):
        wid = lax.axis_index("s") * _NC + lax.axis_index("c")
        row0 = wid * bags_w
        pltpu.sync_copy(gidx_hbm.at[pl.ds(row0, bags_w)], idx_v)
        pltpu.sync_copy(goff_hbm.at[pl.ds(row0, bags_w)], off_v)

        lane = jnp.arange(_LANES, dtype=jnp.int32)

        def accum(rows_ref, b):
            bvec = jnp.full((_LANES,), b, dtype=jnp.int32)
            acc0 = jnp.zeros((_LANES,), jnp.float32)
            acc1 = jnp.zeros((_LANES,), jnp.float32)
            for j in range(L):
                jvec = jnp.full((_LANES,), j, dtype=jnp.int32)
                off = plsc.load_gather(off_v, [bvec, jvec])
                acc0 = acc0 + plsc.load_gather(rows_ref, [jvec, off + lane])
                acc1 = acc1 + plsc.load_gather(
                    rows_ref, [jvec, off + (lane + _LANES)])
            out_v[b, pl.ds(0, _LANES)] = acc0
            out_v[b, pl.ds(_LANES, _LANES)] = acc1

        def body(g, carry):
            b0 = 2 * g
            b1 = b0 + 1
            c0 = pltpu.async_copy(p_hbm.at[idx_v.at[b0]], rows0, sem0)
            c1 = pltpu.async_copy(p_hbm.at[idx_v.at[b1]], rows1, sem1)
            c0.wait()
            accum(rows0, b0)
            c1.wait()
            accum(rows1, b1)
            return carry

        lax.fori_loop(0, bags_w // 2, body, 0)
        pltpu.sync_copy(out_v, out_hbm.at[pl.ds(row0, bags_w)])

    return k(P, gidx, goff)


def _finish_tc(sums, fc_bias2d, L):
    B = sums.shape[0]
    C = fc_bias2d.shape[1]
    blk = 512
    inv = 1.0 / float(L)

    def body(s_ref, b_ref, o_ref):
        logits = s_ref[...][:, :C] * inv + b_ref[...]
        m = jnp.max(logits, axis=1, keepdims=True)
        e = jnp.exp(logits - m)
        lse = jnp.log(jnp.sum(e, axis=1, keepdims=True)) + m
        o_ref[...] = logits - lse

    return pl.pallas_call(
        body,
        grid=(B // blk,),
        in_specs=[
            pl.BlockSpec((blk, 2 * _LANES), lambda i: (i, 0)),
            pl.BlockSpec((1, C), lambda i: (0, 0)),
        ],
        out_specs=pl.BlockSpec((blk, C), lambda i: (i, 0)),
        out_shape=jax.ShapeDtypeStruct((B, C), jnp.float32),
    )(sums, fc_bias2d)


def kernel(text, offsets, emb_weight, fc_weight, fc_bias):
    B = offsets.shape[0]
    T = text.shape[0]
    L = T // B
    V = emb_weight.shape[0]
    C = fc_weight.shape[0]

    P, R = _project_tc(emb_weight.T, fc_weight, V)
    text2d = text.reshape(B, L)
    seg = text2d // R
    gidx = text2d - seg * R
    goff = seg * 32
    sums = _bag_sum_sc(gidx, goff, P)
    return _finish_tc(sums, fc_bias.reshape(1, C), L)


# speedup vs baseline: 3.2945x; 1.6739x over previous
import functools

import jax
import jax.numpy as jnp
from jax import lax
from jax.experimental import pallas as pl
from jax.experimental.pallas import tpu as pltpu
from jax.experimental.pallas import tpu_sc as plsc

_NC = 2
_NS = 16
_NW = _NC * _NS
_LANES = 16

_TB = 4096
_NSEG = 4


def _project_tc(embT, wbd, V):
    D = embT.shape[0]
    nblk = -(-V // (_NSEG * _TB))
    R = nblk * _TB
    last_blk = -(-V // _TB) - 1

    def body(x0_ref, x1_ref, x2_ref, x3_ref, w_ref, o_ref):
        x = jnp.concatenate(
            [x0_ref[...], x1_ref[...], x2_ref[...], x3_ref[...]], axis=0)
        o_ref[...] = lax.dot_general(
            x, w_ref[...], (((0,), (0,)), ((), ())),
            preferred_element_type=jnp.float32)

    specs = [
        pl.BlockSpec((D, _TB), functools.partial(
            lambda s, i: (0, jnp.minimum(s * nblk + i, last_blk)), s))
        for s in range(_NSEG)
    ]
    specs.append(pl.BlockSpec((_NSEG * D, 32 * _NSEG), lambda i: (0, 0)))
    return pl.pallas_call(
        body,
        grid=(nblk,),
        in_specs=specs,
        out_specs=pl.BlockSpec((_TB, 32 * _NSEG), lambda i: (i, 0)),
        out_shape=jax.ShapeDtypeStruct((R, 32 * _NSEG), jnp.float32),
    )(embT, embT, embT, embT, wbd), R


def _bag_sum_sc(gidx2, goff2, P, B, L):
    bags_w = B // _NW
    chunks_w = bags_w // 2
    CL = 2 * L
    mesh = plsc.VectorSubcoreMesh(core_axis_name="c", subcore_axis_name="s")

    @functools.partial(
        pl.kernel,
        out_type=jax.ShapeDtypeStruct((B, 2 * _LANES), jnp.float32),
        mesh=mesh,
        compiler_params=pltpu.CompilerParams(needs_layout_passes=False),
        scratch_types=[
            pltpu.VMEM((chunks_w, CL), jnp.int32),
            pltpu.VMEM((chunks_w, CL), jnp.int32),
            pltpu.VMEM((CL, 128), jnp.float32),
            pltpu.VMEM((CL, 128), jnp.float32),
            pltpu.VMEM((bags_w, 2 * _LANES), jnp.float32),
            pltpu.SemaphoreType.DMA,
            pltpu.SemaphoreType.DMA,
        ],
    )
    def k(p_hbm, gidx_hbm, goff_hbm, out_hbm,
          idx_v, off_v, rowsA, rowsB, out_v, semA, semB):
        wid = lax.axis_index("s") * _NC + lax.axis_index("c")
        crow0 = wid * chunks_w
        pltpu.sync_copy(gidx_hbm.at[pl.ds(crow0, chunks_w)], idx_v)
        pltpu.sync_copy(goff_hbm.at[pl.ds(crow0, chunks_w)], off_v)

        lane = jnp.arange(_LANES, dtype=jnp.int32)

        def accum(rows_ref, c):
            cvec = jnp.full((_LANES,), c, dtype=jnp.int32)
            for half in range(2):
                acc0 = jnp.zeros((_LANES,), jnp.float32)
                acc1 = jnp.zeros((_LANES,), jnp.float32)
                for j in range(half * L, (half + 1) * L):
                    jvec = jnp.full((_LANES,), j, dtype=jnp.int32)
                    off = plsc.load_gather(off_v, [cvec, jvec])
                    acc0 = acc0 + plsc.load_gather(
                        rows_ref, [jvec, off + lane])
                    acc1 = acc1 + plsc.load_gather(
                        rows_ref, [jvec, off + (lane + _LANES)])
                b = 2 * c + half
                out_v[b, pl.ds(0, _LANES)] = acc0
                out_v[b, pl.ds(_LANES, _LANES)] = acc1

        def start(c, buf, sem):
            return pltpu.async_copy(p_hbm.at[idx_v.at[c]], buf, sem)

        start(0, rowsA, semA)

        def body(g, carry):
            c0 = 2 * g
            start(c0 + 1, rowsB, semB)
            pltpu.make_async_copy(p_hbm.at[idx_v.at[c0]], rowsA, semA).wait()
            accum(rowsA, c0)

            @pl.when(c0 + 2 < chunks_w)
            def _():
                start(c0 + 2, rowsA, semA)

            pltpu.make_async_copy(
                p_hbm.at[idx_v.at[c0 + 1]], rowsB, semB).wait()
            accum(rowsB, c0 + 1)
            return carry

        lax.fori_loop(0, chunks_w // 2, body, 0)
        pltpu.sync_copy(out_v, out_hbm.at[pl.ds(wid * bags_w, bags_w)])

    return k(P, gidx2, goff2)


def _finish_tc(sums, fc_bias2d, L):
    B = sums.shape[0]
    C = fc_bias2d.shape[1]
    blk = 512
    inv = 1.0 / float(L)

    def body(s_ref, b_ref, o_ref):
        logits = s_ref[...][:, :C] * inv + b_ref[...]
        m = jnp.max(logits, axis=1, keepdims=True)
        e = jnp.exp(logits - m)
        lse = jnp.log(jnp.sum(e, axis=1, keepdims=True)) + m
        o_ref[...] = logits - lse

    return pl.pallas_call(
        body,
        grid=(B // blk,),
        in_specs=[
            pl.BlockSpec((blk, 2 * _LANES), lambda i: (i, 0)),
            pl.BlockSpec((1, C), lambda i: (0, 0)),
        ],
        out_specs=pl.BlockSpec((blk, C), lambda i: (i, 0)),
        out_shape=jax.ShapeDtypeStruct((B, C), jnp.float32),
    )(sums, fc_bias2d)


def kernel(text, offsets, emb_weight, fc_weight, fc_bias):
    B = offsets.shape[0]
    T = text.shape[0]
    L = T // B
    V = emb_weight.shape[0]
    C = fc_weight.shape[0]

    wpad = jnp.pad(fc_weight.T, ((0, 0), (0, 32 - C)))
    wbd = jnp.kron(jnp.eye(_NSEG, dtype=jnp.float32), wpad)

    P, R = _project_tc(emb_weight.T, wbd, V)
    text2d = text.reshape(B // 2, 2 * L)
    seg = text2d // R
    gidx2 = text2d - seg * R
    goff2 = seg * 32
    sums = _bag_sum_sc(gidx2, goff2, P, B, L)
    return _finish_tc(sums, fc_bias.reshape(1, C), L)


# speedup vs baseline: 3.3618x; 1.0204x over previous
import functools

import jax
import jax.numpy as jnp
from jax import lax
from jax.experimental import pallas as pl
from jax.experimental.pallas import tpu as pltpu
from jax.experimental.pallas import tpu_sc as plsc

_NC = 2
_NS = 16
_NW = _NC * _NS
_LANES = 16

_TB = 4096
_NSEG = 4


def _project_tc(embT, wbd, V):
    D = embT.shape[0]
    nblk = -(-V // (_NSEG * _TB))
    R = nblk * _TB
    last_blk = -(-V // _TB) - 1

    def body(x0_ref, x1_ref, x2_ref, x3_ref, w_ref, o_ref):
        x = jnp.concatenate(
            [x0_ref[...], x1_ref[...], x2_ref[...], x3_ref[...]], axis=0)
        o_ref[...] = lax.dot_general(
            x.astype(jnp.bfloat16), w_ref[...].astype(jnp.bfloat16),
            (((0,), (0,)), ((), ())),
            preferred_element_type=jnp.float32)

    specs = [
        pl.BlockSpec((D, _TB), functools.partial(
            lambda s, i: (0, jnp.minimum(s * nblk + i, last_blk)), s))
        for s in range(_NSEG)
    ]
    specs.append(pl.BlockSpec((_NSEG * D, 32 * _NSEG), lambda i: (0, 0)))
    return pl.pallas_call(
        body,
        grid=(nblk,),
        in_specs=specs,
        out_specs=pl.BlockSpec((_TB, 32 * _NSEG), lambda i: (i, 0)),
        out_shape=jax.ShapeDtypeStruct((R, 32 * _NSEG), jnp.float32),
    )(embT, embT, embT, embT, wbd), R


def _bag_sum_sc(gidx2, goff2, P, B, L):
    bags_w = B // _NW
    chunks_w = bags_w // 2
    CL = 2 * L
    mesh = plsc.VectorSubcoreMesh(core_axis_name="c", subcore_axis_name="s")

    @functools.partial(
        pl.kernel,
        out_type=jax.ShapeDtypeStruct((B, 2 * _LANES), jnp.float32),
        mesh=mesh,
        compiler_params=pltpu.CompilerParams(needs_layout_passes=False),
        scratch_types=[
            pltpu.VMEM((chunks_w, CL), jnp.int32),
            pltpu.VMEM((chunks_w, CL), jnp.int32),
            pltpu.VMEM((CL, 128), jnp.float32),
            pltpu.VMEM((CL, 128), jnp.float32),
            pltpu.VMEM((bags_w, 2 * _LANES), jnp.float32),
            pltpu.SemaphoreType.DMA,
            pltpu.SemaphoreType.DMA,
        ],
    )
    def k(p_hbm, gidx_hbm, goff_hbm, out_hbm,
          idx_v, off_v, rowsA, rowsB, out_v, semA, semB):
        wid = lax.axis_index("s") * _NC + lax.axis_index("c")
        crow0 = wid * chunks_w
        pltpu.sync_copy(gidx_hbm.at[pl.ds(crow0, chunks_w)], idx_v)
        pltpu.sync_copy(goff_hbm.at[pl.ds(crow0, chunks_w)], off_v)

        lane = jnp.arange(_LANES, dtype=jnp.int32)

        def accum(rows_ref, c):
            cvec = jnp.full((_LANES,), c, dtype=jnp.int32)
            for half in range(2):
                acc0 = jnp.zeros((_LANES,), jnp.float32)
                acc1 = jnp.zeros((_LANES,), jnp.float32)
                for j in range(half * L, (half + 1) * L):
                    jvec = jnp.full((_LANES,), j, dtype=jnp.int32)
                    off = plsc.load_gather(off_v, [cvec, jvec])
                    acc0 = acc0 + plsc.load_gather(
                        rows_ref, [jvec, off + lane])
                    acc1 = acc1 + plsc.load_gather(
                        rows_ref, [jvec, off + (lane + _LANES)])
                b = 2 * c + half
                out_v[b, pl.ds(0, _LANES)] = acc0
                out_v[b, pl.ds(_LANES, _LANES)] = acc1

        def start(c, buf, sem):
            return pltpu.async_copy(p_hbm.at[idx_v.at[c]], buf, sem)

        start(0, rowsA, semA)

        def body(g, carry):
            c0 = 2 * g
            start(c0 + 1, rowsB, semB)
            pltpu.make_async_copy(p_hbm.at[idx_v.at[c0]], rowsA, semA).wait()
            accum(rowsA, c0)

            @pl.when(c0 + 2 < chunks_w)
            def _():
                start(c0 + 2, rowsA, semA)

            pltpu.make_async_copy(
                p_hbm.at[idx_v.at[c0 + 1]], rowsB, semB).wait()
            accum(rowsB, c0 + 1)
            return carry

        lax.fori_loop(0, chunks_w // 2, body, 0)
        pltpu.sync_copy(out_v, out_hbm.at[pl.ds(wid * bags_w, bags_w)])

    return k(P, gidx2, goff2)


def _finish_tc(sums, fc_bias2d, L):
    B = sums.shape[0]
    C = fc_bias2d.shape[1]
    blk = 512
    inv = 1.0 / float(L)

    def body(s_ref, b_ref, o_ref):
        logits = s_ref[...][:, :C] * inv + b_ref[...]
        m = jnp.max(logits, axis=1, keepdims=True)
        e = jnp.exp(logits - m)
        lse = jnp.log(jnp.sum(e, axis=1, keepdims=True)) + m
        o_ref[...] = logits - lse

    return pl.pallas_call(
        body,
        grid=(B // blk,),
        in_specs=[
            pl.BlockSpec((blk, 2 * _LANES), lambda i: (i, 0)),
            pl.BlockSpec((1, C), lambda i: (0, 0)),
        ],
        out_specs=pl.BlockSpec((blk, C), lambda i: (i, 0)),
        out_shape=jax.ShapeDtypeStruct((B, C), jnp.float32),
    )(sums, fc_bias2d)


def kernel(text, offsets, emb_weight, fc_weight, fc_bias):
    B = offsets.shape[0]
    T = text.shape[0]
    L = T // B
    V = emb_weight.shape[0]
    C = fc_weight.shape[0]

    wpad = jnp.pad(fc_weight.T, ((0, 0), (0, 32 - C)))
    wbd = jnp.kron(jnp.eye(_NSEG, dtype=jnp.float32), wpad)

    P, R = _project_tc(emb_weight.T, wbd, V)
    text2d = text.reshape(B // 2, 2 * L)
    seg = text2d // R
    gidx2 = text2d - seg * R
    goff2 = seg * 32
    sums = _bag_sum_sc(gidx2, goff2, P, B, L)
    return _finish_tc(sums, fc_bias.reshape(1, C), L)


# speedup vs baseline: 3.5380x; 1.0524x over previous
import functools

import jax
import jax.numpy as jnp
from jax import lax
from jax.experimental import pallas as pl
from jax.experimental.pallas import tpu as pltpu
from jax.experimental.pallas import tpu_sc as plsc

_NC = 2
_NS = 16
_NW = _NC * _NS
_LANES = 16

_TB = 4096
_HT = _TB // 2
_NSEG = 4


def _rne_bf16_bits(u):
    return (u + 0x7FFF + ((u >> 16) & 1)) >> 16


def _project_tc(embT, wbd, V):
    D = embT.shape[0]
    nblk = -(-V // (_NSEG * _TB))
    R = nblk * _TB
    last_blk = -(-V // _TB) - 1

    def body(x0_ref, x1_ref, x2_ref, x3_ref, w_ref, o_ref):
        x = jnp.concatenate(
            [x0_ref[...], x1_ref[...], x2_ref[...], x3_ref[...]], axis=0)
        o = lax.dot_general(
            x.astype(jnp.bfloat16), w_ref[...],
            (((0,), (0,)), ((), ())),
            preferred_element_type=jnp.float32)
        eb = _rne_bf16_bits(pltpu.bitcast(o[:_HT, :], jnp.uint32))
        ob = _rne_bf16_bits(pltpu.bitcast(o[_HT:, :], jnp.uint32))
        o_ref[...] = pltpu.bitcast(eb | (ob << 16), jnp.int32)

    specs = [
        pl.BlockSpec((D, _TB), functools.partial(
            lambda s, i: (0, jnp.minimum(s * nblk + i, last_blk)), s))
        for s in range(_NSEG)
    ]
    specs.append(pl.BlockSpec((_NSEG * D, 32 * _NSEG), lambda i: (0, 0)))
    return pl.pallas_call(
        body,
        grid=(nblk,),
        in_specs=specs,
        out_specs=pl.BlockSpec((_HT, 32 * _NSEG), lambda i: (i, 0)),
        out_shape=jax.ShapeDtypeStruct((nblk * _HT, 32 * _NSEG), jnp.int32),
    )(embT, embT, embT, embT, wbd), R


def _bag_sum_sc(gidx2, gmeta2, P, B, L):
    bags_w = B // _NW
    chunks_w = bags_w // 2
    CL = 2 * L
    mesh = plsc.VectorSubcoreMesh(core_axis_name="c", subcore_axis_name="s")

    @functools.partial(
        pl.kernel,
        out_type=jax.ShapeDtypeStruct((B, 2 * _LANES), jnp.float32),
        mesh=mesh,
        compiler_params=pltpu.CompilerParams(needs_layout_passes=False),
        scratch_types=[
            pltpu.VMEM((chunks_w, CL), jnp.int32),
            pltpu.VMEM((chunks_w, CL), jnp.int32),
            pltpu.VMEM((CL, 128), jnp.int32),
            pltpu.VMEM((CL, 128), jnp.int32),
            pltpu.VMEM((bags_w, 2 * _LANES), jnp.float32),
            pltpu.SemaphoreType.DMA,
            pltpu.SemaphoreType.DMA,
        ],
    )
    def k(p_hbm, gidx_hbm, gmeta_hbm, out_hbm,
          idx_v, meta_v, rowsA, rowsB, out_v, semA, semB):
        wid = lax.axis_index("s") * _NC + lax.axis_index("c")
        crow0 = wid * chunks_w
        pltpu.sync_copy(gidx_hbm.at[pl.ds(crow0, chunks_w)], idx_v)
        pltpu.sync_copy(gmeta_hbm.at[pl.ds(crow0, chunks_w)], meta_v)

        lane = jnp.arange(_LANES, dtype=jnp.int32)

        def accum(rows_ref, c):
            cvec = jnp.full((_LANES,), c, dtype=jnp.int32)
            for half in range(2):
                acc0 = jnp.zeros((_LANES,), jnp.float32)
                acc1 = jnp.zeros((_LANES,), jnp.float32)
                for j in range(half * L, (half + 1) * L):
                    jvec = jnp.full((_LANES,), j, dtype=jnp.int32)
                    meta = plsc.load_gather(meta_v, [cvec, jvec])
                    off = (meta & 3) << 5
                    par = (meta & 4) > 0
                    w0 = plsc.load_gather(rows_ref, [jvec, off + lane])
                    w1 = plsc.load_gather(
                        rows_ref, [jvec, off + (lane + _LANES)])
                    a0, b0 = plsc.unpack(
                        plsc.bitcast(w0, jnp.bfloat16),
                        format=plsc.PackFormat.INTERLEAVED)
                    a1, b1 = plsc.unpack(
                        plsc.bitcast(w1, jnp.bfloat16),
                        format=plsc.PackFormat.INTERLEAVED)
                    acc0 = acc0 + jnp.where(par, b0, a0)
                    acc1 = acc1 + jnp.where(par, b1, a1)
                b = 2 * c + half
                out_v[b, pl.ds(0, _LANES)] = acc0
                out_v[b, pl.ds(_LANES, _LANES)] = acc1

        def start(c, buf, sem):
            return pltpu.async_copy(p_hbm.at[idx_v.at[c]], buf, sem)

        start(0, rowsA, semA)

        def body(g, carry):
            c0 = 2 * g
            start(c0 + 1, rowsB, semB)
            pltpu.make_async_copy(p_hbm.at[idx_v.at[c0]], rowsA, semA).wait()
            accum(rowsA, c0)

            @pl.when(c0 + 2 < chunks_w)
            def _():
                start(c0 + 2, rowsA, semA)

            pltpu.make_async_copy(
                p_hbm.at[idx_v.at[c0 + 1]], rowsB, semB).wait()
            accum(rowsB, c0 + 1)
            return carry

        lax.fori_loop(0, chunks_w // 2, body, 0)
        pltpu.sync_copy(out_v, out_hbm.at[pl.ds(wid * bags_w, bags_w)])

    return k(P, gidx2, gmeta2)


def _finish_tc(sums, fc_bias2d, L):
    B = sums.shape[0]
    C = fc_bias2d.shape[1]
    blk = 512
    inv = 1.0 / float(L)

    def body(s_ref, b_ref, o_ref):
        logits = s_ref[...][:, :C] * inv + b_ref[...]
        m = jnp.max(logits, axis=1, keepdims=True)
        e = jnp.exp(logits - m)
        lse = jnp.log(jnp.sum(e, axis=1, keepdims=True)) + m
        o_ref[...] = logits - lse

    return pl.pallas_call(
        body,
        grid=(B // blk,),
        in_specs=[
            pl.BlockSpec((blk, 2 * _LANES), lambda i: (i, 0)),
            pl.BlockSpec((1, C), lambda i: (0, 0)),
        ],
        out_specs=pl.BlockSpec((blk, C), lambda i: (i, 0)),
        out_shape=jax.ShapeDtypeStruct((B, C), jnp.float32),
    )(sums, fc_bias2d)


def kernel(text, offsets, emb_weight, fc_weight, fc_bias):
    B = offsets.shape[0]
    T = text.shape[0]
    L = T // B
    V = emb_weight.shape[0]
    C = fc_weight.shape[0]

    wpad = jnp.pad(fc_weight.T, ((0, 0), (0, 32 - C)))
    wbd = jnp.kron(
        jnp.eye(_NSEG, dtype=jnp.float32), wpad).astype(jnp.bfloat16)

    P, R = _project_tc(emb_weight.T, wbd, V)

    text2d = text.reshape(B // 2, 2 * L)
    seg = text2d // R
    rr = text2d - seg * R
    blk_i = rr // _TB
    pos = rr - blk_i * _TB
    par = (pos >= _HT).astype(jnp.int32)
    gidx2 = blk_i * _HT + (pos - par * _HT)
    gmeta2 = seg | (par << 2)

    sums = _bag_sum_sc(gidx2, gmeta2, P, B, L)
    return _finish_tc(sums, fc_bias.reshape(1, C), L)


# speedup vs baseline: 3.9560x; 1.1181x over previous
import functools

import jax
import jax.numpy as jnp
from jax import lax
from jax.experimental import pallas as pl
from jax.experimental.pallas import tpu as pltpu
from jax.experimental.pallas import tpu_sc as plsc

_NC = 2
_NS = 16
_NW = _NC * _NS
_LANES = 16

_TB = 4096
_HT = _TB // 2
_NSEG = 4


def _rne_bf16_bits(u):
    return (u + 0x7FFF + ((u >> 16) & 1)) >> 16


def _project_tc(embT, wbd, V):
    D = embT.shape[0]
    nblk = -(-V // (_NSEG * _TB))
    R = nblk * _TB
    last_blk = -(-V // _TB) - 1

    def body(x0_ref, x1_ref, x2_ref, x3_ref, w_ref, o_ref):
        x = jnp.concatenate(
            [x0_ref[...], x1_ref[...], x2_ref[...], x3_ref[...]], axis=0)
        o = lax.dot_general(
            x.astype(jnp.bfloat16), w_ref[...],
            (((0,), (0,)), ((), ())),
            preferred_element_type=jnp.float32)
        eb = _rne_bf16_bits(pltpu.bitcast(o[:_HT, :], jnp.uint32))
        ob = _rne_bf16_bits(pltpu.bitcast(o[_HT:, :], jnp.uint32))
        o_ref[...] = pltpu.bitcast(eb | (ob << 16), jnp.int32)

    specs = [
        pl.BlockSpec((D, _TB), functools.partial(
            lambda s, i: (0, jnp.minimum(s * nblk + i, last_blk)), s))
        for s in range(_NSEG)
    ]
    specs.append(pl.BlockSpec((_NSEG * D, 32 * _NSEG), lambda i: (0, 0)))
    return pl.pallas_call(
        body,
        grid=(nblk,),
        in_specs=specs,
        out_specs=pl.BlockSpec((_HT, 32 * _NSEG), lambda i: (i, 0)),
        out_shape=jax.ShapeDtypeStruct((nblk * _HT, 32 * _NSEG), jnp.int32),
    )(embT, embT, embT, embT, wbd), R


def _bag_sum_sc(gidx2, gpar2, P3, B, L):
    bags_w = B // _NW
    chunks_w = bags_w // 2
    CL = 2 * L
    mesh = plsc.VectorSubcoreMesh(core_axis_name="c", subcore_axis_name="s")

    @functools.partial(
        pl.kernel,
        out_type=jax.ShapeDtypeStruct((B, 2 * _LANES), jnp.float32),
        mesh=mesh,
        compiler_params=pltpu.CompilerParams(
            needs_layout_passes=False, use_tc_tiling_on_sc=False),
        scratch_types=[
            pltpu.VMEM((chunks_w, CL), jnp.int32),
            pltpu.VMEM((chunks_w, CL), jnp.int32),
            pltpu.VMEM((CL, 32), jnp.int32),
            pltpu.VMEM((CL, 32), jnp.int32),
            pltpu.VMEM((bags_w, 2 * _LANES), jnp.float32),
            pltpu.SemaphoreType.DMA,
            pltpu.SemaphoreType.DMA,
        ],
    )
    def k(p_hbm, gidx_hbm, gpar_hbm, out_hbm,
          idx_v, par_v, rowsA, rowsB, out_v, semA, semB):
        wid = lax.axis_index("s") * _NC + lax.axis_index("c")
        crow0 = wid * chunks_w
        pltpu.sync_copy(gidx_hbm.at[pl.ds(crow0, chunks_w)], idx_v)
        pltpu.sync_copy(gpar_hbm.at[pl.ds(crow0, chunks_w)], par_v)

        def accum(rows_ref, c):
            cvec = jnp.full((_LANES,), c, dtype=jnp.int32)
            for half in range(2):
                acc0 = jnp.zeros((_LANES,), jnp.float32)
                acc1 = jnp.zeros((_LANES,), jnp.float32)
                for j in range(half * L, (half + 1) * L):
                    jvec = jnp.full((_LANES,), j, dtype=jnp.int32)
                    par = plsc.load_gather(par_v, [cvec, jvec]) > 0
                    w0 = rows_ref[j, pl.ds(0, _LANES)]
                    w1 = rows_ref[j, pl.ds(_LANES, _LANES)]
                    a0, b0 = plsc.unpack(
                        plsc.bitcast(w0, jnp.bfloat16),
                        format=plsc.PackFormat.INTERLEAVED)
                    a1, b1 = plsc.unpack(
                        plsc.bitcast(w1, jnp.bfloat16),
                        format=plsc.PackFormat.INTERLEAVED)
                    acc0 = acc0 + jnp.where(par, b0, a0)
                    acc1 = acc1 + jnp.where(par, b1, a1)
                b = 2 * c + half
                out_v[b, pl.ds(0, _LANES)] = acc0
                out_v[b, pl.ds(_LANES, _LANES)] = acc1

        def start(c, buf, sem):
            return pltpu.async_copy(p_hbm.at[idx_v.at[c]], buf, sem)

        start(0, rowsA, semA)

        def body(g, carry):
            c0 = 2 * g
            start(c0 + 1, rowsB, semB)
            pltpu.make_async_copy(p_hbm.at[idx_v.at[c0]], rowsA, semA).wait()
            accum(rowsA, c0)

            @pl.when(c0 + 2 < chunks_w)
            def _():
                start(c0 + 2, rowsA, semA)

            pltpu.make_async_copy(
                p_hbm.at[idx_v.at[c0 + 1]], rowsB, semB).wait()
            accum(rowsB, c0 + 1)
            return carry

        lax.fori_loop(0, chunks_w // 2, body, 0)
        pltpu.sync_copy(out_v, out_hbm.at[pl.ds(wid * bags_w, bags_w)])

    return k(P3, gidx2, gpar2)


def _finish_tc(sums, fc_bias2d, L):
    B = sums.shape[0]
    C = fc_bias2d.shape[1]
    blk = 512
    inv = 1.0 / float(L)

    def body(s_ref, b_ref, o_ref):
        logits = s_ref[...][:, :C] * inv + b_ref[...]
        m = jnp.max(logits, axis=1, keepdims=True)
        e = jnp.exp(logits - m)
        lse = jnp.log(jnp.sum(e, axis=1, keepdims=True)) + m
        o_ref[...] = logits - lse

    return pl.pallas_call(
        body,
        grid=(B // blk,),
        in_specs=[
            pl.BlockSpec((blk, 2 * _LANES), lambda i: (i, 0)),
            pl.BlockSpec((1, C), lambda i: (0, 0)),
        ],
        out_specs=pl.BlockSpec((blk, C), lambda i: (i, 0)),
        out_shape=jax.ShapeDtypeStruct((B, C), jnp.float32),
    )(sums, fc_bias2d)


def kernel(text, offsets, emb_weight, fc_weight, fc_bias):
    B = offsets.shape[0]
    T = text.shape[0]
    L = T // B
    V = emb_weight.shape[0]
    C = fc_weight.shape[0]

    wpad = jnp.pad(fc_weight.T, ((0, 0), (0, 32 - C)))
    wbd = jnp.kron(
        jnp.eye(_NSEG, dtype=jnp.float32), wpad).astype(jnp.bfloat16)

    P, R = _project_tc(emb_weight.T, wbd, V)
    P3 = P.reshape(P.shape[0] * 4, 32)

    text2d = text.reshape(B // 2, 2 * L)
    seg = text2d // R
    rr = text2d - seg * R
    blk_i = rr // _TB
    pos = rr - blk_i * _TB
    par = (pos >= _HT).astype(jnp.int32)
    prow = blk_i * _HT + (pos - par * _HT)
    gidx2 = prow * 4 + seg
    sums = _bag_sum_sc(gidx2, par, P3, B, L)
    return _finish_tc(sums, fc_bias.reshape(1, C), L)


# speedup vs baseline: 4.0159x; 1.0151x over previous
import functools

import jax
import jax.numpy as jnp
from jax import lax
from jax.experimental import pallas as pl
from jax.experimental.pallas import tpu as pltpu
from jax.experimental.pallas import tpu_sc as plsc

_NC = 2
_NS = 16
_NW = _NC * _NS
_LANES = 16

_TB = 4096
_HT = _TB // 2
_NSEG = 4


def _rne_bf16_bits(u):
    return (u + 0x7FFF + ((u >> 16) & 1)) >> 16


def _project_tc(embT, wbd, V):
    D = embT.shape[0]
    nblk = -(-V // (_NSEG * _TB))
    R = nblk * _TB
    last_blk = -(-V // _TB) - 1

    def body(x0_ref, x1_ref, x2_ref, x3_ref, w_ref, o_ref):
        x = jnp.concatenate(
            [x0_ref[...], x1_ref[...], x2_ref[...], x3_ref[...]], axis=0)
        o = lax.dot_general(
            x.astype(jnp.bfloat16), w_ref[...],
            (((0,), (0,)), ((), ())),
            preferred_element_type=jnp.float32)
        eb = _rne_bf16_bits(pltpu.bitcast(o[:_HT, :], jnp.uint32))
        ob = _rne_bf16_bits(pltpu.bitcast(o[_HT:, :], jnp.uint32))
        o_ref[...] = pltpu.bitcast(eb | (ob << 16), jnp.int32)

    specs = [
        pl.BlockSpec((D, _TB), functools.partial(
            lambda s, i: (0, jnp.minimum(s * nblk + i, last_blk)), s))
        for s in range(_NSEG)
    ]
    specs.append(pl.BlockSpec((_NSEG * D, 32 * _NSEG), lambda i: (0, 0)))
    return pl.pallas_call(
        body,
        grid=(nblk,),
        in_specs=specs,
        out_specs=pl.BlockSpec((_HT, 32 * _NSEG), lambda i: (i, 0)),
        out_shape=jax.ShapeDtypeStruct((nblk * _HT, 32 * _NSEG), jnp.int32),
    )(embT, embT, embT, embT, wbd), R


def _bag_sum_sc(gidx2, gpar2, P3, B, L):
    bags_w = B // _NW
    chunks_w = bags_w // 2
    CL = 2 * L
    mesh = plsc.VectorSubcoreMesh(core_axis_name="c", subcore_axis_name="s")

    @functools.partial(
        pl.kernel,
        out_type=jax.ShapeDtypeStruct((B, 2 * _LANES), jnp.float32),
        mesh=mesh,
        compiler_params=pltpu.CompilerParams(
            needs_layout_passes=False, use_tc_tiling_on_sc=False),
        scratch_types=[
            pltpu.VMEM((chunks_w, CL), jnp.int32),
            pltpu.VMEM((chunks_w, CL), jnp.int32),
            pltpu.VMEM((CL, 32), jnp.int32),
            pltpu.VMEM((CL, 32), jnp.int32),
            pltpu.VMEM((bags_w, 2 * _LANES), jnp.float32),
            pltpu.SemaphoreType.DMA,
            pltpu.SemaphoreType.DMA,
        ],
    )
    def k(p_hbm, gidx_hbm, gpar_hbm, out_hbm,
          idx_v, par_v, rowsA, rowsB, out_v, semA, semB):
        wid = lax.axis_index("s") * _NC + lax.axis_index("c")
        crow0 = wid * chunks_w
        pltpu.sync_copy(gidx_hbm.at[pl.ds(crow0, chunks_w)], idx_v)
        pltpu.sync_copy(gpar_hbm.at[pl.ds(crow0, chunks_w)], par_v)

        def accum(rows_ref, c):
            cvec = jnp.full((_LANES,), c, dtype=jnp.int32)
            for half in range(2):
                acc0 = jnp.zeros((_LANES,), jnp.float32)
                acc1 = jnp.zeros((_LANES,), jnp.float32)
                for j in range(half * L, (half + 1) * L):
                    jvec = jnp.full((_LANES,), j, dtype=jnp.int32)
                    par = plsc.load_gather(par_v, [cvec, jvec]) > 0
                    w0 = rows_ref[j, pl.ds(0, _LANES)]
                    w1 = rows_ref[j, pl.ds(_LANES, _LANES)]
                    a0, b0 = plsc.unpack(
                        plsc.bitcast(w0, jnp.bfloat16),
                        format=plsc.PackFormat.INTERLEAVED)
                    a1, b1 = plsc.unpack(
                        plsc.bitcast(w1, jnp.bfloat16),
                        format=plsc.PackFormat.INTERLEAVED)
                    acc0 = acc0 + jnp.where(par, b0, a0)
                    acc1 = acc1 + jnp.where(par, b1, a1)
                b = 2 * c + half
                out_v[b, pl.ds(0, _LANES)] = acc0
                out_v[b, pl.ds(_LANES, _LANES)] = acc1

        def start(c, buf, sem):
            return pltpu.async_copy(p_hbm.at[idx_v.at[c]], buf, sem)

        start(0, rowsA, semA)

        def body(g, carry):
            c0 = 2 * g
            start(c0 + 1, rowsB, semB)
            pltpu.make_async_copy(p_hbm.at[idx_v.at[c0]], rowsA, semA).wait()
            accum(rowsA, c0)

            @pl.when(c0 + 2 < chunks_w)
            def _():
                start(c0 + 2, rowsA, semA)

            pltpu.make_async_copy(
                p_hbm.at[idx_v.at[c0 + 1]], rowsB, semB).wait()
            accum(rowsB, c0 + 1)
            return carry

        lax.fori_loop(0, chunks_w // 2, body, 0)
        pltpu.sync_copy(out_v, out_hbm.at[pl.ds(wid * bags_w, bags_w)])

    return k(P3, gidx2, gpar2)


def _finish_tc(sums, fc_bias2d, L):
    B = sums.shape[0]
    C = fc_bias2d.shape[1]
    blk = B
    inv = 1.0 / float(L)

    def body(s_ref, b_ref, o_ref):
        logits = s_ref[...][:, :C] * inv + b_ref[...]
        m = jnp.max(logits, axis=1, keepdims=True)
        e = jnp.exp(logits - m)
        lse = jnp.log(jnp.sum(e, axis=1, keepdims=True)) + m
        o_ref[...] = logits - lse

    return pl.pallas_call(
        body,
        grid=(B // blk,),
        in_specs=[
            pl.BlockSpec((blk, 2 * _LANES), lambda i: (i, 0)),
            pl.BlockSpec((1, C), lambda i: (0, 0)),
        ],
        out_specs=pl.BlockSpec((blk, C), lambda i: (i, 0)),
        out_shape=jax.ShapeDtypeStruct((B, C), jnp.float32),
    )(sums, fc_bias2d)


def kernel(text, offsets, emb_weight, fc_weight, fc_bias):
    B = offsets.shape[0]
    T = text.shape[0]
    L = T // B
    V = emb_weight.shape[0]
    C = fc_weight.shape[0]

    wpad = jnp.pad(fc_weight.T, ((0, 0), (0, 32 - C)))
    wbd = jnp.kron(
        jnp.eye(_NSEG, dtype=jnp.float32), wpad).astype(jnp.bfloat16)

    P, R = _project_tc(emb_weight.T, wbd, V)
    P3 = P.reshape(P.shape[0] * 4, 32)

    text2d = text.reshape(B // 2, 2 * L)
    seg = text2d // R
    rr = text2d - seg * R
    blk_i = rr // _TB
    pos = rr - blk_i * _TB
    par = (pos >= _HT).astype(jnp.int32)
    prow = blk_i * _HT + (pos - par * _HT)
    gidx2 = prow * 4 + seg
    sums = _bag_sum_sc(gidx2, par, P3, B, L)
    return _finish_tc(sums, fc_bias.reshape(1, C), L)
